# Initial kernel scaffold; baseline (speedup 1.0000x reference)
import jax
import jax.numpy as jnp
from jax.experimental import pallas as pl


def kernel(x, edge_index, W1, b1, W2, b2):
    raise NotImplementedError("write your pallas kernel here")


# speedup vs baseline: 34.7982x; 34.7982x over previous
import functools

import jax
import jax.numpy as jnp
from jax import lax
from jax.experimental import pallas as pl
from jax.experimental.pallas import tpu as pltpu
from jax.experimental.pallas import tpu_sc as plsc

_B = 128
_K = 8
_NC = 2
_NS = 16


@functools.lru_cache(maxsize=None)
def _make_scatter(N, E, D):
    NW = _NC * _NS
    n_units = E // (_K * _B)
    assert E % (_K * _B) == 0
    per_tile = n_units // NW
    extra = n_units - per_tile * NW
    rpt = (((N + _NS - 1) // _NS) + 7) // 8 * 8
    Npad = rpt * _NS

    mesh = plsc.VectorSubcoreMesh(core_axis_name="c", subcore_axis_name="s")

    @functools.partial(
        pl.kernel,
        out_type=jax.ShapeDtypeStruct((_NC, Npad, D), jnp.float32),
        mesh=mesh,
        compiler_params=pltpu.CompilerParams(use_tc_tiling_on_sc=False),
        scratch_types=[
            pltpu.VMEM((_K, _B), jnp.int32),
            pltpu.VMEM((_K, _B), jnp.int32),
            pltpu.VMEM((_K, _B, D), jnp.float32),
            pltpu.VMEM_SHARED((Npad, D), jnp.float32),
            pltpu.SemaphoreType.DMA,
        ],
    )
    def scatter_kernel(x_hbm, e_hbm, z_hbm, out_hbm, src_v, dst_v, rows_v, acc, sem):
        c = lax.axis_index("c")
        s = lax.axis_index("s")
        wid = c * _NS + s

        r0 = s * rpt
        pltpu.sync_copy(z_hbm, acc.at[pl.ds(r0, rpt)])
        plsc.subcore_barrier()

        def do_unit(u):
            pltpu.sync_copy(e_hbm.at[0, u], src_v)
            pltpu.sync_copy(e_hbm.at[1, u], dst_v)
            descs = [
                pltpu.async_copy(x_hbm.at[src_v.at[j]], rows_v.at[j], sem)
                for j in range(_K)
            ]
            for d in descs:
                d.wait()
            for j in range(_K):
                pltpu.sync_copy(rows_v.at[j], acc.at[dst_v.at[j]], add=True)

        def chunk_body(ci, carry):
            do_unit(wid * per_tile + ci)
            return carry

        lax.fori_loop(0, per_tile, chunk_body, 0)

        @pl.when(wid < extra)
        def _():
            do_unit(NW * per_tile + wid)

        plsc.subcore_barrier()

        pltpu.sync_copy(acc.at[pl.ds(r0, rpt)], out_hbm.at[c, pl.ds(r0, rpt)])

    return scatter_kernel, Npad, rpt


@functools.lru_cache(maxsize=None)
def _make_combine(N, Npad, D, relu):
    BN = 2000
    assert N % BN == 0

    def body(p_ref, w_ref, b_ref, o_ref):
        sm = p_ref[0] + p_ref[1]
        h = jnp.dot(sm, w_ref[:], preferred_element_type=jnp.float32) + b_ref[:]
        o_ref[:] = jnp.maximum(h, 0.0) if relu else h

    return pl.pallas_call(
        body,
        grid=(N // BN,),
        in_specs=[
            pl.BlockSpec((2, BN, D), lambda i: (0, i, 0)),
            pl.BlockSpec((D, D), lambda i: (0, 0)),
            pl.BlockSpec((1, D), lambda i: (0, 0)),
        ],
        out_specs=pl.BlockSpec((BN, D), lambda i: (i, 0)),
        out_shape=jax.ShapeDtypeStruct((N, D), jnp.float32),
    )


def kernel(x, edge_index, W1, b1, W2, b2):
    N, D = x.shape
    E = edge_index.shape[1]
    e4 = edge_index.reshape(2, E // (_K * _B), _K, _B)

    scatter, Npad, rpt = _make_scatter(N, E, D)
    zeros = jnp.zeros((rpt, D), jnp.float32)

    p1 = scatter(x, e4, zeros)
    h1 = _make_combine(N, Npad, D, True)(p1, W1, b1.reshape(1, D))
    p2 = scatter(h1, e4, zeros)
    out = _make_combine(N, Npad, D, False)(p2, W2, b2.reshape(1, D))
    return out
